# Initial kernel scaffold; baseline (speedup 1.0000x reference)
import functools

import jax
import jax.numpy as jnp
from jax import lax
from jax.experimental import pallas as pl
from jax.experimental.pallas import tpu as pltpu
from jax.experimental.pallas import tpu_sc as plsc

N = 10000
E = 320000
D = 128
NC = 2
NS = 16
NW = NC * NS
EPW = E // NW
CH = 80
NCHUNK = EPW // CH
NP = 10240
RPS = N // NS

_mesh = plsc.VectorSubcoreMesh(core_axis_name="c", subcore_axis_name="s")


def _degree_hist(ei2d):
    epb = E // NS
    cpb = NP // NS

    @functools.partial(
        pl.kernel,
        out_type=jax.ShapeDtypeStruct((2, NP), jnp.float32),
        mesh=_mesh,
        scratch_types=[
            pltpu.VMEM((epb,), jnp.int32),
            pltpu.VMEM((NP,), jnp.float32),
            pltpu.VMEM((NS, cpb), jnp.float32),
            pltpu.VMEM((cpb,), jnp.float32),
            pltpu.VMEM_SHARED((NS, NP), jnp.float32),
        ],
    )
    def k(ei_hbm, out_hbm, idx_v, hist_v, tmp_v, acc_v, sh):
        c = lax.axis_index("c")
        s = lax.axis_index("s")
        zeros16 = jnp.zeros((16,), jnp.float32)
        ones16 = jnp.ones((16,), jnp.float32)

        @pl.loop(0, NP, step=16)
        def _(i):
            hist_v[pl.ds(i, 16)] = zeros16

        pltpu.sync_copy(ei_hbm.at[c, pl.ds(s * epb, epb)], idx_v)

        @pl.loop(0, epb, step=16)
        def _(i):
            v = idx_v[pl.ds(i, 16)]
            plsc.addupdate_scatter(hist_v, [v], ones16)

        pltpu.sync_copy(hist_v, sh.at[s])
        plsc.subcore_barrier()

        cs = s * cpb
        pltpu.sync_copy(sh.at[:, pl.ds(cs, cpb)], tmp_v)

        @pl.loop(0, cpb, step=16)
        def _(i):
            a = tmp_v[0, pl.ds(i, 16)]
            for t in range(1, NS):
                a = a + tmp_v[t, pl.ds(i, 16)]
            acc_v[pl.ds(i, 16)] = a

        pltpu.sync_copy(acc_v, out_hbm.at[c, pl.ds(cs, cpb)])

    return k(ei2d)


def _aggregate(hs, src_flat, dst2d, zrows):
    @functools.partial(
        pl.kernel,
        out_type=jax.ShapeDtypeStruct((2, N, D), jnp.float32),
        mesh=_mesh,
        scratch_types=[
            pltpu.VMEM((EPW,), jnp.int32),
            pltpu.VMEM((NCHUNK, CH), jnp.int32),
            pltpu.VMEM((CH, D), jnp.float32),
            pltpu.VMEM_SHARED((N, D), jnp.float32),
            pltpu.SemaphoreType.DMA,
        ],
    )
    def k(hs_hbm, src_hbm, dst_hbm, z_hbm, out_hbm, src_v, dst_v, rows_v, acc_sh, sem):
        c = lax.axis_index("c")
        s = lax.axis_index("s")
        wid = s * NC + c
        rb = s * RPS

        @pl.when(c == 0)
        def _():
            pltpu.sync_copy(hs_hbm.at[pl.ds(rb, RPS)], acc_sh.at[pl.ds(rb, RPS)])

        @pl.when(c == 1)
        def _():
            pltpu.sync_copy(z_hbm, acc_sh.at[pl.ds(rb, RPS)])

        pltpu.sync_copy(src_hbm.at[pl.ds(wid * EPW, EPW)], src_v)
        pltpu.sync_copy(dst_hbm.at[pl.ds(wid * NCHUNK, NCHUNK)], dst_v)
        plsc.subcore_barrier()

        @pl.loop(0, NCHUNK)
        def _(j):
            pltpu.async_copy(hs_hbm.at[src_v.at[pl.ds(j * CH, CH)]], rows_v, sem).wait()
            pltpu.sync_copy(rows_v, acc_sh.at[dst_v.at[j]], add=True)

        plsc.subcore_barrier()
        pltpu.sync_copy(acc_sh.at[pl.ds(rb, RPS)], out_hbm.at[c, pl.ds(rb, RPS)])

    return k(hs, src_flat, dst2d, zrows)


_ROWS_BLK = 1000


def _prep(x, dg):
    def body(x_ref, dg_ref, hs_ref, inv_ref):
        inv = lax.rsqrt(dg_ref[...] + 1.0)
        inv_ref[...] = inv
        hs_ref[...] = x_ref[...] * inv[:, 0:1]

    return pl.pallas_call(
        body,
        grid=(N // _ROWS_BLK,),
        in_specs=[
            pl.BlockSpec((_ROWS_BLK, D), lambda i: (i, 0)),
            pl.BlockSpec((_ROWS_BLK, 2), lambda i: (i, 0)),
        ],
        out_specs=[
            pl.BlockSpec((_ROWS_BLK, D), lambda i: (i, 0)),
            pl.BlockSpec((_ROWS_BLK, 2), lambda i: (i, 0)),
        ],
        out_shape=[
            jax.ShapeDtypeStruct((N, D), jnp.float32),
            jax.ShapeDtypeStruct((N, 2), jnp.float32),
        ],
    )(x, dg)


def _layer_tc(p0, p1, invs, w_bf16, b2d, last):
    def body(p0_ref, p1_ref, inv_ref, w_ref, b_ref, o_ref):
        inv = inv_ref[...]
        agg = (p0_ref[...] + p1_ref[...]) * inv[:, 1:2]
        y = jnp.dot(agg.astype(jnp.bfloat16), w_ref[...],
                    preferred_element_type=jnp.float32) + b_ref[...]
        if last:
            o_ref[...] = y
        else:
            o_ref[...] = jnp.maximum(y, 0.0) * inv[:, 0:1]

    return pl.pallas_call(
        body,
        grid=(N // _ROWS_BLK,),
        in_specs=[
            pl.BlockSpec((_ROWS_BLK, D), lambda i: (i, 0)),
            pl.BlockSpec((_ROWS_BLK, D), lambda i: (i, 0)),
            pl.BlockSpec((_ROWS_BLK, 2), lambda i: (i, 0)),
            pl.BlockSpec((D, D), lambda i: (0, 0)),
            pl.BlockSpec((1, D), lambda i: (0, 0)),
        ],
        out_specs=pl.BlockSpec((_ROWS_BLK, D), lambda i: (i, 0)),
        out_shape=jax.ShapeDtypeStruct((N, D), jnp.float32),
    )(p0, p1, invs, w_bf16, b2d)


def kernel(x, edge_index, W1, b1, W2, b2, W3, b3):
    ei = edge_index.astype(jnp.int32)
    src_flat = ei[0]
    dst2d = ei[1].reshape(E // CH, CH)
    zrows = jnp.zeros((RPS, D), jnp.float32)

    degs = _degree_hist(ei)
    dg = jnp.stack([degs[0, :N], degs[1, :N]], axis=1)
    hs, invs = _prep(x, dg)

    for W, b, last in ((W1, b1, False), (W2, b2, False), (W3, b3, True)):
        part = _aggregate(hs, src_flat, dst2d, zrows)
        hs = _layer_tc(part[0], part[1], invs, W.astype(jnp.bfloat16),
                       b.reshape(1, D), last)
    return hs


# speedup vs baseline: 10.5924x; 10.5924x over previous
import dataclasses
import functools

import jax
import jax.numpy as jnp
from jax import lax
from jax.experimental import pallas as pl
from jax.experimental.pallas import tpu as pltpu
from jax.experimental.pallas import tpu_sc as plsc

N = 10000
E = 320000
D = 128
NC = 2
NS = 16
NW = NC * NS
EPW = E // NW
CH = 80
NCHUNK = EPW // CH
NP = 10240
RPS = NP // NS

_mesh = plsc.VectorSubcoreMesh(core_axis_name="c", subcore_axis_name="s")

_sc_params = pltpu.CompilerParams()
if "needs_layout_passes" in pltpu.CompilerParams.__dataclass_fields__:
    _sc_params = dataclasses.replace(_sc_params, needs_layout_passes=False)


def _degree_hist(ei_flat):
    epb = E // NS
    cpb = NP // NS

    @functools.partial(
        pl.kernel,
        out_type=jax.ShapeDtypeStruct((2 * NP,), jnp.float32),
        mesh=_mesh,
        compiler_params=_sc_params,
        scratch_types=[
            pltpu.VMEM((epb,), jnp.int32),
            pltpu.VMEM((NP,), jnp.float32),
            pltpu.VMEM((NS, cpb), jnp.float32),
            pltpu.VMEM((cpb,), jnp.float32),
            pltpu.VMEM_SHARED((NS, NP), jnp.float32),
        ],
    )
    def k(ei_hbm, out_hbm, idx_v, hist_v, tmp_v, acc_v, sh):
        c = lax.axis_index("c")
        s = lax.axis_index("s")
        zeros16 = jnp.zeros((16,), jnp.float32)
        ones16 = jnp.ones((16,), jnp.float32)

        @pl.loop(0, NP, step=16)
        def _(i):
            hist_v[pl.ds(i, 16)] = zeros16

        pltpu.sync_copy(ei_hbm.at[pl.ds(c * E + s * epb, epb)], idx_v)

        @pl.loop(0, epb, step=16)
        def _(i):
            v = idx_v[pl.ds(i, 16)]
            plsc.addupdate_scatter(hist_v, [v], ones16)

        pltpu.sync_copy(hist_v, sh.at[s])
        plsc.subcore_barrier()

        cs = s * cpb
        pltpu.sync_copy(sh.at[:, pl.ds(cs, cpb)], tmp_v)

        @pl.loop(0, cpb, step=16)
        def _(i):
            a = tmp_v[0, pl.ds(i, 16)]
            for t in range(1, NS):
                a = a + tmp_v[t, pl.ds(i, 16)]
            acc_v[pl.ds(i, 16)] = a

        pltpu.sync_copy(acc_v, out_hbm.at[pl.ds(c * NP + cs, cpb)])

    return k(ei_flat)


def _aggregate(hs, src_flat, dst3d, zrows):
    @functools.partial(
        pl.kernel,
        out_type=jax.ShapeDtypeStruct((2, NP, D), jnp.float32),
        mesh=_mesh,
        compiler_params=_sc_params,
        scratch_types=[
            pltpu.VMEM((EPW,), jnp.int32),
            pltpu.VMEM((NCHUNK, CH), jnp.int32),
            pltpu.VMEM((CH, D), jnp.float32),
            pltpu.VMEM_SHARED((NP, D), jnp.float32),
            pltpu.SemaphoreType.DMA,
        ],
    )
    def k(hs_hbm, src_hbm, dst_hbm, z_hbm, out_hbm, src_v, dst_v, rows_v, acc_sh, sem):
        c = lax.axis_index("c")
        s = lax.axis_index("s")
        wid = s * NC + c
        rb = s * RPS

        @pl.when(c == 0)
        def _():
            pltpu.sync_copy(hs_hbm.at[pl.ds(rb, RPS)], acc_sh.at[pl.ds(rb, RPS)])

        @pl.when(c == 1)
        def _():
            pltpu.sync_copy(z_hbm, acc_sh.at[pl.ds(rb, RPS)])

        pltpu.sync_copy(src_hbm.at[pl.ds(wid * EPW, EPW)], src_v)
        pltpu.sync_copy(dst_hbm.at[wid], dst_v)
        plsc.subcore_barrier()

        @pl.loop(0, NCHUNK)
        def _(j):
            pltpu.async_copy(hs_hbm.at[src_v.at[pl.ds(j * CH, CH)]], rows_v, sem).wait()
            pltpu.sync_copy(rows_v, acc_sh.at[dst_v.at[j]], add=True)

        plsc.subcore_barrier()
        pltpu.sync_copy(acc_sh.at[pl.ds(rb, RPS)], out_hbm.at[c, pl.ds(rb, RPS)])

    return k(hs, src_flat, dst3d, zrows)


_ROWS_BLK = 1024


def _prep(x, dg):
    def body(x_ref, dg_ref, hs_ref, inv_ref):
        inv = lax.rsqrt(dg_ref[...] + 1.0)
        inv_ref[...] = inv
        hs_ref[...] = x_ref[...] * inv[:, 0:1]

    return pl.pallas_call(
        body,
        grid=(NP // _ROWS_BLK,),
        in_specs=[
            pl.BlockSpec((_ROWS_BLK, D), lambda i: (i, 0)),
            pl.BlockSpec((_ROWS_BLK, 2), lambda i: (i, 0)),
        ],
        out_specs=[
            pl.BlockSpec((_ROWS_BLK, D), lambda i: (i, 0)),
            pl.BlockSpec((_ROWS_BLK, 2), lambda i: (i, 0)),
        ],
        out_shape=[
            jax.ShapeDtypeStruct((NP, D), jnp.float32),
            jax.ShapeDtypeStruct((NP, 2), jnp.float32),
        ],
    )(x, dg)


def _layer_tc(p0, p1, invs, w_bf16, b2d, last):
    def body(p0_ref, p1_ref, inv_ref, w_ref, b_ref, o_ref):
        inv = inv_ref[...]
        agg = (p0_ref[...] + p1_ref[...]) * inv[:, 1:2]
        y = jnp.dot(agg.astype(jnp.bfloat16), w_ref[...],
                    preferred_element_type=jnp.float32) + b_ref[...]
        if last:
            o_ref[...] = y
        else:
            o_ref[...] = jnp.maximum(y, 0.0) * inv[:, 0:1]

    return pl.pallas_call(
        body,
        grid=(NP // _ROWS_BLK,),
        in_specs=[
            pl.BlockSpec((_ROWS_BLK, D), lambda i: (i, 0)),
            pl.BlockSpec((_ROWS_BLK, D), lambda i: (i, 0)),
            pl.BlockSpec((_ROWS_BLK, 2), lambda i: (i, 0)),
            pl.BlockSpec((D, D), lambda i: (0, 0)),
            pl.BlockSpec((1, D), lambda i: (0, 0)),
        ],
        out_specs=pl.BlockSpec((_ROWS_BLK, D), lambda i: (i, 0)),
        out_shape=jax.ShapeDtypeStruct((NP, D), jnp.float32),
    )(p0, p1, invs, w_bf16, b2d)


def kernel(x, edge_index, W1, b1, W2, b2, W3, b3):
    ei = edge_index.astype(jnp.int32)
    src_flat = ei[0]
    dst3d = ei[1].reshape(NW, NCHUNK, CH)
    zrows = jnp.zeros((RPS, D), jnp.float32)

    degf = _degree_hist(ei.reshape(-1))
    dg = jnp.stack([degf[:NP], degf[NP:]], axis=1)
    xp = jnp.pad(x, ((0, NP - N), (0, 0)))
    hs, invs = _prep(xp, dg)

    for W, b, last in ((W1, b1, False), (W2, b2, False), (W3, b3, True)):
        part = _aggregate(hs, src_flat, dst3d, zrows)
        hs = _layer_tc(part[0], part[1], invs, W.astype(jnp.bfloat16),
                       b.reshape(1, D), last)
    return hs[:N]


# speedup vs baseline: 13.3302x; 1.2585x over previous
import dataclasses
import functools

import jax
import jax.numpy as jnp
from jax import lax
from jax.experimental import pallas as pl
from jax.experimental.pallas import tpu as pltpu
from jax.experimental.pallas import tpu_sc as plsc

N = 10000
E = 320000
D = 128
NC = 2
NS = 16
NW = NC * NS
EPW = E // NW
CH = 80
NCHUNK = EPW // CH
NP = 10240
RPS = NP // NS

_mesh = plsc.VectorSubcoreMesh(core_axis_name="c", subcore_axis_name="s")

_sc_params = pltpu.CompilerParams()
if "needs_layout_passes" in pltpu.CompilerParams.__dataclass_fields__:
    _sc_params = dataclasses.replace(_sc_params, needs_layout_passes=False)


def _degree_hist(ei_flat):
    epb = E // NS
    cpb = NP // NS

    @functools.partial(
        pl.kernel,
        out_type=jax.ShapeDtypeStruct((2 * NP,), jnp.float32),
        mesh=_mesh,
        compiler_params=_sc_params,
        scratch_types=[
            pltpu.VMEM((epb,), jnp.int32),
            pltpu.VMEM((NP,), jnp.float32),
            pltpu.VMEM((NS, cpb), jnp.float32),
            pltpu.VMEM((cpb,), jnp.float32),
            pltpu.VMEM_SHARED((NS, NP), jnp.float32),
        ],
    )
    def k(ei_hbm, out_hbm, idx_v, hist_v, tmp_v, acc_v, sh):
        c = lax.axis_index("c")
        s = lax.axis_index("s")
        zeros16 = jnp.zeros((16,), jnp.float32)
        ones16 = jnp.ones((16,), jnp.float32)

        @pl.loop(0, NP, step=16)
        def _(i):
            hist_v[pl.ds(i, 16)] = zeros16

        pltpu.sync_copy(ei_hbm.at[pl.ds(c * E + s * epb, epb)], idx_v)

        @pl.loop(0, epb, step=16)
        def _(i):
            v = idx_v[pl.ds(i, 16)]
            plsc.addupdate_scatter(hist_v, [v], ones16)

        pltpu.sync_copy(hist_v, sh.at[s])
        plsc.subcore_barrier()

        cs = s * cpb
        pltpu.sync_copy(sh.at[:, pl.ds(cs, cpb)], tmp_v)

        @pl.loop(0, cpb, step=16)
        def _(i):
            a = tmp_v[0, pl.ds(i, 16)]
            for t in range(1, NS):
                a = a + tmp_v[t, pl.ds(i, 16)]
            acc_v[pl.ds(i, 16)] = a

        pltpu.sync_copy(acc_v, out_hbm.at[pl.ds(c * NP + cs, cpb)])

    return k(ei_flat)


def _aggregate(hs, src_flat, dst3d, zrows):
    @functools.partial(
        pl.kernel,
        out_type=jax.ShapeDtypeStruct((2, NP, D), jnp.float32),
        mesh=_mesh,
        compiler_params=_sc_params,
        scratch_types=[
            pltpu.VMEM((EPW,), jnp.int32),
            pltpu.VMEM((NCHUNK, CH), jnp.int32),
            pltpu.VMEM((CH, D), jnp.float32),
            pltpu.VMEM((CH, D), jnp.float32),
            pltpu.VMEM_SHARED((NP, D), jnp.float32),
            pltpu.SemaphoreType.DMA,
            pltpu.SemaphoreType.DMA,
        ],
    )
    def k(hs_hbm, src_hbm, dst_hbm, z_hbm, out_hbm,
          src_v, dst_v, rows_a, rows_b, acc_sh, sem_a, sem_b):
        c = lax.axis_index("c")
        s = lax.axis_index("s")
        wid = s * NC + c
        rb = s * RPS

        @pl.when(c == 0)
        def _():
            pltpu.sync_copy(hs_hbm.at[pl.ds(rb, RPS)], acc_sh.at[pl.ds(rb, RPS)])

        @pl.when(c == 1)
        def _():
            pltpu.sync_copy(z_hbm, acc_sh.at[pl.ds(rb, RPS)])

        pltpu.sync_copy(src_hbm.at[pl.ds(wid * EPW, EPW)], src_v)
        pltpu.sync_copy(dst_hbm.at[wid], dst_v)
        plsc.subcore_barrier()

        def gather(j, buf, sem):
            return pltpu.async_copy(hs_hbm.at[src_v.at[pl.ds(j * CH, CH)]], buf, sem)

        gather(0, rows_a, sem_a)

        @pl.loop(0, NCHUNK - 1, step=2)
        def _(j):
            pltpu.make_async_copy(hs_hbm.at[src_v.at[pl.ds(j * CH, CH)]], rows_a, sem_a).wait()
            gather(j + 1, rows_b, sem_b)
            pltpu.sync_copy(rows_a, acc_sh.at[dst_v.at[j]], add=True)
            pltpu.make_async_copy(hs_hbm.at[src_v.at[pl.ds((j + 1) * CH, CH)]], rows_b, sem_b).wait()
            gather(j + 2, rows_a, sem_a)
            pltpu.sync_copy(rows_b, acc_sh.at[dst_v.at[j + 1]], add=True)

        pltpu.make_async_copy(
            hs_hbm.at[src_v.at[pl.ds((NCHUNK - 1) * CH, CH)]], rows_a, sem_a).wait()
        pltpu.sync_copy(rows_a, acc_sh.at[dst_v.at[NCHUNK - 1]], add=True)

        plsc.subcore_barrier()
        pltpu.sync_copy(acc_sh.at[pl.ds(rb, RPS)], out_hbm.at[c, pl.ds(rb, RPS)])

    return k(hs, src_flat, dst3d, zrows)


_ROWS_BLK = 1024


def _prep(x, dg):
    def body(x_ref, dg_ref, hs_ref, inv_ref):
        inv = lax.rsqrt(dg_ref[...] + 1.0)
        inv_ref[...] = inv
        hs_ref[...] = x_ref[...] * inv[:, 0:1]

    return pl.pallas_call(
        body,
        grid=(NP // _ROWS_BLK,),
        in_specs=[
            pl.BlockSpec((_ROWS_BLK, D), lambda i: (i, 0)),
            pl.BlockSpec((_ROWS_BLK, 2), lambda i: (i, 0)),
        ],
        out_specs=[
            pl.BlockSpec((_ROWS_BLK, D), lambda i: (i, 0)),
            pl.BlockSpec((_ROWS_BLK, 2), lambda i: (i, 0)),
        ],
        out_shape=[
            jax.ShapeDtypeStruct((NP, D), jnp.float32),
            jax.ShapeDtypeStruct((NP, 2), jnp.float32),
        ],
    )(x, dg)


def _layer_tc(p0, p1, invs, w_bf16, b2d, last):
    def body(p0_ref, p1_ref, inv_ref, w_ref, b_ref, o_ref):
        inv = inv_ref[...]
        agg = (p0_ref[...] + p1_ref[...]) * inv[:, 1:2]
        y = jnp.dot(agg.astype(jnp.bfloat16), w_ref[...],
                    preferred_element_type=jnp.float32) + b_ref[...]
        if last:
            o_ref[...] = y
        else:
            o_ref[...] = jnp.maximum(y, 0.0) * inv[:, 0:1]

    return pl.pallas_call(
        body,
        grid=(NP // _ROWS_BLK,),
        in_specs=[
            pl.BlockSpec((_ROWS_BLK, D), lambda i: (i, 0)),
            pl.BlockSpec((_ROWS_BLK, D), lambda i: (i, 0)),
            pl.BlockSpec((_ROWS_BLK, 2), lambda i: (i, 0)),
            pl.BlockSpec((D, D), lambda i: (0, 0)),
            pl.BlockSpec((1, D), lambda i: (0, 0)),
        ],
        out_specs=pl.BlockSpec((_ROWS_BLK, D), lambda i: (i, 0)),
        out_shape=jax.ShapeDtypeStruct((NP, D), jnp.float32),
    )(p0, p1, invs, w_bf16, b2d)


def kernel(x, edge_index, W1, b1, W2, b2, W3, b3):
    ei = edge_index.astype(jnp.int32)
    src_flat = ei[0]
    dst3d = ei[1].reshape(NW, NCHUNK, CH)
    zrows = jnp.zeros((RPS, D), jnp.float32)

    degf = _degree_hist(ei.reshape(-1))
    dg = jnp.stack([degf[:NP], degf[NP:]], axis=1)
    xp = jnp.pad(x, ((0, NP - N), (0, 0)))
    hs, invs = _prep(xp, dg)

    for W, b, last in ((W1, b1, False), (W2, b2, False), (W3, b3, True)):
        part = _aggregate(hs, src_flat, dst3d, zrows)
        hs = _layer_tc(part[0], part[1], invs, W.astype(jnp.bfloat16),
                       b.reshape(1, D), last)
    return hs[:N]
